# baseline (speedup 1.0000x reference)
import jax
import jax.numpy as jnp
from jax.experimental import pallas as pl
from jax.experimental.pallas import tpu as pltpu

_BN = 512
_COMMITMENT_COST = 0.25


def _vq_block(z_ref, w_ref, out_ref, loss_ref):
    i = pl.program_id(0)
    z = z_ref[...]
    w = w_ref[...]
    k = w.shape[0]

    a = jnp.sum(z * z, axis=1, keepdims=True)
    b = jnp.sum(w * w, axis=1)
    c = jax.lax.dot_general(z, w, (((1,), (1,)), ((), ())),
                            preferred_element_type=jnp.float32)
    dist = (a + b[None, :]) - 2.0 * c

    min_d = jnp.min(dist, axis=1, keepdims=True)
    kiota = jax.lax.broadcasted_iota(jnp.int32, dist.shape, 1)
    idx = jnp.min(jnp.where(dist == min_d, kiota, k), axis=1, keepdims=True)

    onehot = (kiota == idx).astype(jnp.float32)
    zq = jax.lax.dot_general(onehot, w, (((1,), (0,)), ((), ())),
                             preferred_element_type=jnp.float32,
                             precision=jax.lax.Precision.HIGHEST)

    d = zq - z
    out_ref[...] = z + d

    @pl.when(i == 0)
    def _():
        loss_ref[0, 0] = 0.0
    loss_ref[0, 0] += jnp.sum(d * d)


def kernel(z, W):
    n, dim = z.shape
    k = W.shape[0]
    grid = n // _BN
    out, loss_sum = pl.pallas_call(
        _vq_block,
        grid=(grid,),
        in_specs=[
            pl.BlockSpec((_BN, dim), lambda i: (i, 0)),
            pl.BlockSpec((k, dim), lambda i: (0, 0)),
        ],
        out_specs=[
            pl.BlockSpec((_BN, dim), lambda i: (i, 0)),
            pl.BlockSpec(block_shape=(1, 1), index_map=lambda i: (0, 0),
                         memory_space=pltpu.SMEM),
        ],
        out_shape=[
            jax.ShapeDtypeStruct((n, dim), jnp.float32),
            jax.ShapeDtypeStruct((1, 1), jnp.float32),
        ],
    )(z, W)
    loss = loss_sum[0, 0] * ((1.0 + _COMMITMENT_COST) / (n * dim))
    return out, loss


# speedup vs baseline: 1.5656x; 1.5656x over previous
import jax
import jax.numpy as jnp
from jax.experimental import pallas as pl
from jax.experimental.pallas import tpu as pltpu

_BN = 512
_COMMITMENT_COST = 0.25


def _vq_block(z_ref, w_ref, out_ref, loss_ref):
    i = pl.program_id(0)
    z = z_ref[...]
    w = w_ref[...]
    k = w.shape[0]

    a = jnp.sum(z * z, axis=1, keepdims=True)
    b = jnp.sum(w * w, axis=1)
    c = jax.lax.dot_general(z, w, (((1,), (1,)), ((), ())),
                            preferred_element_type=jnp.float32)
    dist = (a + b[None, :]) - 2.0 * c

    min_d = jnp.min(dist, axis=1, keepdims=True)
    kiota = jax.lax.broadcasted_iota(jnp.int32, dist.shape, 1)
    idx = jnp.min(jnp.where(dist == min_d, kiota, k), axis=1, keepdims=True)

    onehot = (kiota == idx).astype(jnp.bfloat16)
    zq = jax.lax.dot_general(onehot, w.astype(jnp.bfloat16),
                             (((1,), (0,)), ((), ())),
                             preferred_element_type=jnp.float32)

    d = zq - z
    out_ref[...] = z + d

    @pl.when(i == 0)
    def _():
        loss_ref[0, 0] = 0.0
    loss_ref[0, 0] += jnp.sum(d * d)


def kernel(z, W):
    n, dim = z.shape
    k = W.shape[0]
    grid = n // _BN
    out, loss_sum = pl.pallas_call(
        _vq_block,
        grid=(grid,),
        in_specs=[
            pl.BlockSpec((_BN, dim), lambda i: (i, 0)),
            pl.BlockSpec((k, dim), lambda i: (0, 0)),
        ],
        out_specs=[
            pl.BlockSpec((_BN, dim), lambda i: (i, 0)),
            pl.BlockSpec(block_shape=(1, 1), index_map=lambda i: (0, 0),
                         memory_space=pltpu.SMEM),
        ],
        out_shape=[
            jax.ShapeDtypeStruct((n, dim), jnp.float32),
            jax.ShapeDtypeStruct((1, 1), jnp.float32),
        ],
    )(z, W)
    loss = loss_sum[0, 0] * ((1.0 + _COMMITMENT_COST) / (n * dim))
    return out, loss


# speedup vs baseline: 1.8701x; 1.1945x over previous
import jax
import jax.numpy as jnp
from jax.experimental import pallas as pl
from jax.experimental.pallas import tpu as pltpu

_BN = 1024
_COMMITMENT_COST = 0.25


def _vq_block(z_ref, w_ref, out_ref, loss_ref, bt_ref, w2_ref, wbf_ref,
              kio_ref):
    i = pl.program_id(0)
    z = z_ref[...]
    kf = float(w_ref.shape[0])

    @pl.when(i == 0)
    def _():
        w = w_ref[...]
        bt_ref[...] = jnp.sum(w * w, axis=1)[None, :]
        w2_ref[...] = w * (-2.0)
        wbf_ref[...] = w.astype(jnp.bfloat16)
        kio_ref[...] = jax.lax.broadcasted_iota(
            jnp.int32, kio_ref.shape, 1).astype(jnp.float32)
        loss_ref[0, 0] = 0.0

    c2 = jax.lax.dot_general(z, w2_ref[...], (((1,), (1,)), ((), ())),
                             preferred_element_type=jnp.float32)
    a = jnp.sum(z * z, axis=1, keepdims=True)
    dist = (a + bt_ref[...]) + c2

    min_d = jnp.min(dist, axis=1, keepdims=True)
    kiota = kio_ref[...]
    idx = jnp.min(jnp.where(dist == min_d, kiota, kf), axis=1, keepdims=True)

    onehot = (kiota == idx).astype(jnp.bfloat16)
    zq = jax.lax.dot_general(onehot, wbf_ref[...], (((1,), (0,)), ((), ())),
                             preferred_element_type=jnp.float32)

    d = zq - z
    out_ref[...] = z + d
    loss_ref[0, 0] += jnp.sum(d * d)


def kernel(z, W):
    n, dim = z.shape
    k = W.shape[0]
    grid = n // _BN
    out, loss_sum = pl.pallas_call(
        _vq_block,
        grid=(grid,),
        in_specs=[
            pl.BlockSpec((_BN, dim), lambda i: (i, 0)),
            pl.BlockSpec((k, dim), lambda i: (0, 0)),
        ],
        out_specs=[
            pl.BlockSpec((_BN, dim), lambda i: (i, 0)),
            pl.BlockSpec(block_shape=(1, 1), index_map=lambda i: (0, 0),
                         memory_space=pltpu.SMEM),
        ],
        out_shape=[
            jax.ShapeDtypeStruct((n, dim), jnp.float32),
            jax.ShapeDtypeStruct((1, 1), jnp.float32),
        ],
        scratch_shapes=[
            pltpu.VMEM((1, k), jnp.float32),
            pltpu.VMEM((k, dim), jnp.float32),
            pltpu.VMEM((k, dim), jnp.bfloat16),
            pltpu.VMEM((1, k), jnp.float32),
        ],
    )(z, W)
    loss = loss_sum[0, 0] * ((1.0 + _COMMITMENT_COST) / (n * dim))
    return out, loss


# speedup vs baseline: 1.9890x; 1.0636x over previous
import jax
import jax.numpy as jnp
from jax.experimental import pallas as pl
from jax.experimental.pallas import tpu as pltpu

_BN = 2048
_COMMITMENT_COST = 0.25


def _vq_block(z_ref, w_ref, out_ref, loss_ref, bt_ref, w2_ref, wbf_ref,
              kio_ref):
    i = pl.program_id(0)
    z = z_ref[...]
    kf = float(w_ref.shape[0])

    @pl.when(i == 0)
    def _():
        w = w_ref[...]
        bt_ref[...] = jnp.sum(w * w, axis=1)[None, :]
        w2_ref[...] = w * (-2.0)
        wbf_ref[...] = w.astype(jnp.bfloat16)
        kio_ref[...] = jax.lax.broadcasted_iota(
            jnp.int32, kio_ref.shape, 1).astype(jnp.float32)
        loss_ref[0, 0] = 0.0

    c2 = jax.lax.dot_general(z, w2_ref[...], (((1,), (1,)), ((), ())),
                             preferred_element_type=jnp.float32)
    a = jnp.sum(z * z, axis=1, keepdims=True)
    dist = (a + bt_ref[...]) + c2

    min_d = jnp.min(dist, axis=1, keepdims=True)
    kiota = kio_ref[...]
    idx = jnp.min(jnp.where(dist == min_d, kiota, kf), axis=1, keepdims=True)

    onehot = (kiota == idx).astype(jnp.bfloat16)
    zq = jax.lax.dot_general(onehot, wbf_ref[...], (((1,), (0,)), ((), ())),
                             preferred_element_type=jnp.float32)

    d = zq - z
    out_ref[...] = z + d
    loss_ref[0, 0] += jnp.sum(d * d)


def kernel(z, W):
    n, dim = z.shape
    k = W.shape[0]
    grid = n // _BN
    out, loss_sum = pl.pallas_call(
        _vq_block,
        grid=(grid,),
        in_specs=[
            pl.BlockSpec((_BN, dim), lambda i: (i, 0)),
            pl.BlockSpec((k, dim), lambda i: (0, 0)),
        ],
        out_specs=[
            pl.BlockSpec((_BN, dim), lambda i: (i, 0)),
            pl.BlockSpec(block_shape=(1, 1), index_map=lambda i: (0, 0),
                         memory_space=pltpu.SMEM),
        ],
        out_shape=[
            jax.ShapeDtypeStruct((n, dim), jnp.float32),
            jax.ShapeDtypeStruct((1, 1), jnp.float32),
        ],
        scratch_shapes=[
            pltpu.VMEM((1, k), jnp.float32),
            pltpu.VMEM((k, dim), jnp.float32),
            pltpu.VMEM((k, dim), jnp.bfloat16),
            pltpu.VMEM((1, k), jnp.float32),
        ],
    )(z, W)
    loss = loss_sum[0, 0] * ((1.0 + _COMMITMENT_COST) / (n * dim))
    return out, loss


# speedup vs baseline: 2.0226x; 1.0169x over previous
import jax
import jax.numpy as jnp
from jax.experimental import pallas as pl
from jax.experimental.pallas import tpu as pltpu

_BN = 3072
_COMMITMENT_COST = 0.25


def _vq_block(z_ref, w_ref, out_ref, loss_ref, bt_ref, w2_ref, wbf_ref,
              kio_ref):
    i = pl.program_id(0)
    z = z_ref[...]
    kf = float(w_ref.shape[0])

    @pl.when(i == 0)
    def _():
        w = w_ref[...]
        bt_ref[...] = jnp.sum(w * w, axis=1)[None, :]
        w2_ref[...] = w * (-2.0)
        wbf_ref[...] = w.astype(jnp.bfloat16)
        kio_ref[...] = jax.lax.broadcasted_iota(
            jnp.int32, kio_ref.shape, 1).astype(jnp.float32)
        loss_ref[0, 0] = 0.0

    c2 = jax.lax.dot_general(z, w2_ref[...], (((1,), (1,)), ((), ())),
                             preferred_element_type=jnp.float32)
    a = jnp.sum(z * z, axis=1, keepdims=True)
    dist = (a + bt_ref[...]) + c2

    min_d = jnp.min(dist, axis=1, keepdims=True)
    kiota = kio_ref[...]
    idx = jnp.min(jnp.where(dist == min_d, kiota, kf), axis=1, keepdims=True)

    onehot = (kiota == idx).astype(jnp.bfloat16)
    zq = jax.lax.dot_general(onehot, wbf_ref[...], (((1,), (0,)), ((), ())),
                             preferred_element_type=jnp.float32)

    d = zq - z
    out_ref[...] = z + d
    loss_ref[0, 0] += jnp.sum(d * d)


def kernel(z, W):
    n, dim = z.shape
    k = W.shape[0]
    grid = n // _BN
    out, loss_sum = pl.pallas_call(
        _vq_block,
        grid=(grid,),
        in_specs=[
            pl.BlockSpec((_BN, dim), lambda i: (i, 0)),
            pl.BlockSpec((k, dim), lambda i: (0, 0)),
        ],
        out_specs=[
            pl.BlockSpec((_BN, dim), lambda i: (i, 0)),
            pl.BlockSpec(block_shape=(1, 1), index_map=lambda i: (0, 0),
                         memory_space=pltpu.SMEM),
        ],
        out_shape=[
            jax.ShapeDtypeStruct((n, dim), jnp.float32),
            jax.ShapeDtypeStruct((1, 1), jnp.float32),
        ],
        scratch_shapes=[
            pltpu.VMEM((1, k), jnp.float32),
            pltpu.VMEM((k, dim), jnp.float32),
            pltpu.VMEM((k, dim), jnp.bfloat16),
            pltpu.VMEM((1, k), jnp.float32),
        ],
    )(z, W)
    loss = loss_sum[0, 0] * ((1.0 + _COMMITMENT_COST) / (n * dim))
    return out, loss


# speedup vs baseline: 2.2597x; 1.1172x over previous
import jax
import jax.numpy as jnp
from jax.experimental import pallas as pl
from jax.experimental.pallas import tpu as pltpu

_BN = 3072
_COMMITMENT_COST = 0.25


def _vq_block(z_ref, w_ref, out_ref, loss_ref, bt_ref, w2_ref, wbf_ref,
              kio_ref):
    i = pl.program_id(0)
    z = z_ref[...]
    kf = float(w_ref.shape[0])

    @pl.when(i == 0)
    def _():
        w = w_ref[...]
        bt_ref[...] = jnp.sum(w * w, axis=1)[None, :]
        w2_ref[...] = w * (-2.0)
        wbf_ref[...] = w.astype(jnp.bfloat16)
        kio_ref[...] = jax.lax.broadcasted_iota(
            jnp.int32, kio_ref.shape, 1).astype(jnp.float32)
        loss_ref[0, 0] = 0.0

    w2 = w2_ref[...]
    bt = bt_ref[...]
    kiota = kio_ref[...]
    wbf = wbf_ref[...]

    def half(zh):
        c2 = jax.lax.dot_general(zh, w2, (((1,), (1,)), ((), ())),
                                 preferred_element_type=jnp.float32)
        a = jnp.sum(zh * zh, axis=1, keepdims=True)
        dist = (a + bt) + c2
        min_d = jnp.min(dist, axis=1, keepdims=True)
        idx = jnp.min(jnp.where(dist == min_d, kiota, kf), axis=1,
                      keepdims=True)
        onehot = (kiota == idx).astype(jnp.bfloat16)
        zq = jax.lax.dot_general(onehot, wbf, (((1,), (0,)), ((), ())),
                                 preferred_element_type=jnp.float32)
        d = zq - zh
        return zh + d, jnp.sum(d * d)

    h = _BN // 2
    o1, l1 = half(z[:h, :])
    o2, l2 = half(z[h:, :])
    out_ref[:h, :] = o1
    out_ref[h:, :] = o2
    loss_ref[0, 0] += l1 + l2


def kernel(z, W):
    n, dim = z.shape
    k = W.shape[0]
    grid = n // _BN
    out, loss_sum = pl.pallas_call(
        _vq_block,
        grid=(grid,),
        in_specs=[
            pl.BlockSpec((_BN, dim), lambda i: (i, 0)),
            pl.BlockSpec((k, dim), lambda i: (0, 0)),
        ],
        out_specs=[
            pl.BlockSpec((_BN, dim), lambda i: (i, 0)),
            pl.BlockSpec(block_shape=(1, 1), index_map=lambda i: (0, 0),
                         memory_space=pltpu.SMEM),
        ],
        out_shape=[
            jax.ShapeDtypeStruct((n, dim), jnp.float32),
            jax.ShapeDtypeStruct((1, 1), jnp.float32),
        ],
        scratch_shapes=[
            pltpu.VMEM((1, k), jnp.float32),
            pltpu.VMEM((k, dim), jnp.float32),
            pltpu.VMEM((k, dim), jnp.bfloat16),
            pltpu.VMEM((1, k), jnp.float32),
        ],
    )(z, W)
    loss = loss_sum[0, 0] * ((1.0 + _COMMITMENT_COST) / (n * dim))
    return out, loss


# speedup vs baseline: 2.4850x; 1.0997x over previous
import jax
import jax.numpy as jnp
from jax.experimental import pallas as pl
from jax.experimental.pallas import tpu as pltpu

_BN = 3072
_COMMITMENT_COST = 0.25
_SLICES = 4


def _vq_block(z_ref, w_ref, out_ref, loss_ref, bt_ref, w2_ref, wbf_ref,
              kio_ref):
    i = pl.program_id(0)
    z = z_ref[...]
    kf = float(w_ref.shape[0])

    @pl.when(i == 0)
    def _():
        w = w_ref[...]
        bt_ref[...] = jnp.sum(w * w, axis=1)[None, :]
        w2_ref[...] = w * (-2.0)
        wbf_ref[...] = w.astype(jnp.bfloat16)
        kio_ref[...] = jax.lax.broadcasted_iota(
            jnp.int32, kio_ref.shape, 1).astype(jnp.float32)
        loss_ref[0, 0] = 0.0

    w2 = w2_ref[...]
    bt = bt_ref[...]
    kiota = kio_ref[...]
    wbf = wbf_ref[...]

    def half(zh):
        c2 = jax.lax.dot_general(zh, w2, (((1,), (1,)), ((), ())),
                                 preferred_element_type=jnp.float32)
        a = jnp.sum(zh * zh, axis=1, keepdims=True)
        dist = (a + bt) + c2
        min_d = jnp.min(dist, axis=1, keepdims=True)
        idx = jnp.min(jnp.where(dist == min_d, kiota, kf), axis=1,
                      keepdims=True)
        onehot = (kiota == idx).astype(jnp.bfloat16)
        zq = jax.lax.dot_general(onehot, wbf, (((1,), (0,)), ((), ())),
                                 preferred_element_type=jnp.float32)
        d = zq - zh
        return zh + d, jnp.sum(d * d)

    h = _BN // _SLICES
    parts = [half(z[s * h:(s + 1) * h, :]) for s in range(_SLICES)]
    for s, (o, _) in enumerate(parts):
        out_ref[s * h:(s + 1) * h, :] = o
    loss_ref[0, 0] += sum(l for _, l in parts)


def kernel(z, W):
    n, dim = z.shape
    k = W.shape[0]
    grid = n // _BN
    out, loss_sum = pl.pallas_call(
        _vq_block,
        grid=(grid,),
        in_specs=[
            pl.BlockSpec((_BN, dim), lambda i: (i, 0)),
            pl.BlockSpec((k, dim), lambda i: (0, 0)),
        ],
        out_specs=[
            pl.BlockSpec((_BN, dim), lambda i: (i, 0)),
            pl.BlockSpec(block_shape=(1, 1), index_map=lambda i: (0, 0),
                         memory_space=pltpu.SMEM),
        ],
        out_shape=[
            jax.ShapeDtypeStruct((n, dim), jnp.float32),
            jax.ShapeDtypeStruct((1, 1), jnp.float32),
        ],
        scratch_shapes=[
            pltpu.VMEM((1, k), jnp.float32),
            pltpu.VMEM((k, dim), jnp.float32),
            pltpu.VMEM((k, dim), jnp.bfloat16),
            pltpu.VMEM((1, k), jnp.float32),
        ],
    )(z, W)
    loss = loss_sum[0, 0] * ((1.0 + _COMMITMENT_COST) / (n * dim))
    return out, loss


# speedup vs baseline: 2.6203x; 1.0544x over previous
import jax
import jax.numpy as jnp
from jax.experimental import pallas as pl
from jax.experimental.pallas import tpu as pltpu

_BN = 3072
_COMMITMENT_COST = 0.25
_SLICES = 6


def _vq_block(z_ref, w_ref, out_ref, loss_ref, bt_ref, w2_ref, wbf_ref,
              kio_ref):
    i = pl.program_id(0)
    z = z_ref[...]
    kf = float(w_ref.shape[0])

    @pl.when(i == 0)
    def _():
        w = w_ref[...]
        bt_ref[...] = jnp.sum(w * w, axis=1)[None, :]
        w2_ref[...] = w * (-2.0)
        wbf_ref[...] = w.astype(jnp.bfloat16)
        kio_ref[...] = jax.lax.broadcasted_iota(
            jnp.int32, kio_ref.shape, 1).astype(jnp.float32)
        loss_ref[0, 0] = 0.0

    w2 = w2_ref[...]
    bt = bt_ref[...]
    kiota = kio_ref[...]
    wbf = wbf_ref[...]

    def half(zh):
        c2 = jax.lax.dot_general(zh, w2, (((1,), (1,)), ((), ())),
                                 preferred_element_type=jnp.float32)
        a = jnp.sum(zh * zh, axis=1, keepdims=True)
        dist = (a + bt) + c2
        min_d = jnp.min(dist, axis=1, keepdims=True)
        idx = jnp.min(jnp.where(dist == min_d, kiota, kf), axis=1,
                      keepdims=True)
        onehot = (kiota == idx).astype(jnp.bfloat16)
        zq = jax.lax.dot_general(onehot, wbf, (((1,), (0,)), ((), ())),
                                 preferred_element_type=jnp.float32)
        d = zq - zh
        return zh + d, jnp.sum(d * d)

    h = _BN // _SLICES
    parts = [half(z[s * h:(s + 1) * h, :]) for s in range(_SLICES)]
    for s, (o, _) in enumerate(parts):
        out_ref[s * h:(s + 1) * h, :] = o
    loss_ref[0, 0] += sum(l for _, l in parts)


def kernel(z, W):
    n, dim = z.shape
    k = W.shape[0]
    grid = n // _BN
    out, loss_sum = pl.pallas_call(
        _vq_block,
        grid=(grid,),
        in_specs=[
            pl.BlockSpec((_BN, dim), lambda i: (i, 0)),
            pl.BlockSpec((k, dim), lambda i: (0, 0)),
        ],
        out_specs=[
            pl.BlockSpec((_BN, dim), lambda i: (i, 0)),
            pl.BlockSpec(block_shape=(1, 1), index_map=lambda i: (0, 0),
                         memory_space=pltpu.SMEM),
        ],
        out_shape=[
            jax.ShapeDtypeStruct((n, dim), jnp.float32),
            jax.ShapeDtypeStruct((1, 1), jnp.float32),
        ],
        scratch_shapes=[
            pltpu.VMEM((1, k), jnp.float32),
            pltpu.VMEM((k, dim), jnp.float32),
            pltpu.VMEM((k, dim), jnp.bfloat16),
            pltpu.VMEM((1, k), jnp.float32),
        ],
    )(z, W)
    loss = loss_sum[0, 0] * ((1.0 + _COMMITMENT_COST) / (n * dim))
    return out, loss


# speedup vs baseline: 2.6621x; 1.0159x over previous
import jax
import jax.numpy as jnp
from jax.experimental import pallas as pl
from jax.experimental.pallas import tpu as pltpu

_BN = 3072
_COMMITMENT_COST = 0.25
_SLICES = 8


def _vq_block(z_ref, w_ref, out_ref, loss_ref, bt_ref, w2_ref, wbf_ref,
              kio_ref):
    i = pl.program_id(0)
    z = z_ref[...]
    kf = float(w_ref.shape[0])

    @pl.when(i == 0)
    def _():
        w = w_ref[...]
        bt_ref[...] = jnp.sum(w * w, axis=1)[None, :]
        w2_ref[...] = w * (-2.0)
        wbf_ref[...] = w.astype(jnp.bfloat16)
        kio_ref[...] = jax.lax.broadcasted_iota(
            jnp.int32, kio_ref.shape, 1).astype(jnp.float32)
        loss_ref[0, 0] = 0.0

    w2 = w2_ref[...]
    bt = bt_ref[...]
    kiota = kio_ref[...]
    wbf = wbf_ref[...]

    def half(zh):
        c2 = jax.lax.dot_general(zh, w2, (((1,), (1,)), ((), ())),
                                 preferred_element_type=jnp.float32)
        a = jnp.sum(zh * zh, axis=1, keepdims=True)
        dist = (a + bt) + c2
        min_d = jnp.min(dist, axis=1, keepdims=True)
        idx = jnp.min(jnp.where(dist == min_d, kiota, kf), axis=1,
                      keepdims=True)
        onehot = (kiota == idx).astype(jnp.bfloat16)
        zq = jax.lax.dot_general(onehot, wbf, (((1,), (0,)), ((), ())),
                                 preferred_element_type=jnp.float32)
        d = zq - zh
        return zh + d, jnp.sum(d * d)

    h = _BN // _SLICES
    parts = [half(z[s * h:(s + 1) * h, :]) for s in range(_SLICES)]
    for s, (o, _) in enumerate(parts):
        out_ref[s * h:(s + 1) * h, :] = o
    loss_ref[0, 0] += sum(l for _, l in parts)


def kernel(z, W):
    n, dim = z.shape
    k = W.shape[0]
    grid = n // _BN
    out, loss_sum = pl.pallas_call(
        _vq_block,
        grid=(grid,),
        in_specs=[
            pl.BlockSpec((_BN, dim), lambda i: (i, 0)),
            pl.BlockSpec((k, dim), lambda i: (0, 0)),
        ],
        out_specs=[
            pl.BlockSpec((_BN, dim), lambda i: (i, 0)),
            pl.BlockSpec(block_shape=(1, 1), index_map=lambda i: (0, 0),
                         memory_space=pltpu.SMEM),
        ],
        out_shape=[
            jax.ShapeDtypeStruct((n, dim), jnp.float32),
            jax.ShapeDtypeStruct((1, 1), jnp.float32),
        ],
        scratch_shapes=[
            pltpu.VMEM((1, k), jnp.float32),
            pltpu.VMEM((k, dim), jnp.float32),
            pltpu.VMEM((k, dim), jnp.bfloat16),
            pltpu.VMEM((1, k), jnp.float32),
        ],
    )(z, W)
    loss = loss_sum[0, 0] * ((1.0 + _COMMITMENT_COST) / (n * dim))
    return out, loss
